# Initial kernel scaffold; baseline (speedup 1.0000x reference)
import jax
import jax.numpy as jnp
from jax.experimental import pallas as pl


def kernel(x, edge_index, batch, W1, b1, W2, b2, W3, b3, W_lin, b_lin):
    raise NotImplementedError("write your pallas kernel here")


# speedup vs baseline: 6.8611x; 6.8611x over previous
import functools

import jax
import jax.numpy as jnp
from jax import lax
from jax.experimental import pallas as pl
from jax.experimental.pallas import tpu as pltpu
from jax.experimental.pallas import tpu_sc as plsc

N_NODES = 10000
N_EDGES = 320000
D = 128
N_GRAPHS = 64

NC = 2
NS = 16
NW = NC * NS

NP = 10240
CHUNK = 128
EP = 327680
CH_PER_W = EP // NW // CHUNK
ROWS_PER_TILE = NP // NS

_mesh = plsc.VectorSubcoreMesh(core_axis_name="c", subcore_axis_name="s")


@functools.partial(
    pl.kernel,
    mesh=_mesh,
    out_type=jax.ShapeDtypeStruct((NW, NP), jnp.float32),
    scratch_types=[
        pltpu.VMEM((CH_PER_W, CHUNK), jnp.int32),
        pltpu.VMEM((NP,), jnp.float32),
    ],
    compiler_params=pltpu.CompilerParams(needs_layout_passes=False),
)
def _sc_degree(dst_hbm, parts_hbm, dst_v, acc_v):
    cid = lax.axis_index("c")
    sid = lax.axis_index("s")
    wid = sid * NC + cid
    pltpu.sync_copy(dst_hbm.at[pl.ds(wid * CH_PER_W, CH_PER_W)], dst_v)

    def zero(i, carry):
        acc_v[pl.ds(i * 16, 16)] = jnp.zeros((16,), jnp.float32)
        return carry

    lax.fori_loop(0, NP // 16, zero, 0)

    ones = jnp.ones((16,), jnp.float32)

    def count_row(r, carry):
        for c in range(CHUNK // 16):
            idx = dst_v[r, pl.ds(c * 16, 16)]
            plsc.addupdate_scatter(acc_v, [idx], ones)
        return carry

    lax.fori_loop(0, CH_PER_W, count_row, 0)
    pltpu.sync_copy(acc_v, parts_hbm.at[wid])


@functools.partial(
    pl.kernel,
    mesh=_mesh,
    out_type=jax.ShapeDtypeStruct((NC, NP, D), jnp.float32),
    scratch_types=[
        pltpu.VMEM((CH_PER_W, CHUNK), jnp.int32),
        pltpu.VMEM((CH_PER_W, CHUNK), jnp.int32),
        pltpu.VMEM((CHUNK, D), jnp.float32),
        pltpu.VMEM_SHARED((NP, D), jnp.float32),
        pltpu.SemaphoreType.DMA,
    ],
    compiler_params=pltpu.CompilerParams(needs_layout_passes=False),
)
def _sc_edge_scatter(g_hbm, src_hbm, dst_hbm, out_hbm, src_v, dst_v, rows_v,
                     acc_sh, sem):
    cid = lax.axis_index("c")
    sid = lax.axis_index("s")
    wid = sid * NC + cid
    pltpu.sync_copy(src_hbm.at[pl.ds(wid * CH_PER_W, CH_PER_W)], src_v)
    pltpu.sync_copy(dst_hbm.at[pl.ds(wid * CH_PER_W, CH_PER_W)], dst_v)

    def zero_row(r, carry):
        for c in range(D // 16):
            rows_v[r, pl.ds(c * 16, 16)] = jnp.zeros((16,), jnp.float32)
        return carry

    lax.fori_loop(0, CHUNK, zero_row, 0)
    for k in range(ROWS_PER_TILE // CHUNK):
        pltpu.sync_copy(
            rows_v, acc_sh.at[pl.ds(sid * ROWS_PER_TILE + k * CHUNK, CHUNK)])
    plsc.subcore_barrier()

    def body(j, carry):
        pltpu.async_copy(g_hbm.at[src_v.at[j]], rows_v, sem).wait()
        pltpu.sync_copy(rows_v, acc_sh.at[dst_v.at[j]], add=True)
        return carry

    lax.fori_loop(0, CH_PER_W, body, 0)
    plsc.subcore_barrier()
    pltpu.sync_copy(acc_sh.at[pl.ds(sid * ROWS_PER_TILE, ROWS_PER_TILE)],
                    out_hbm.at[cid, pl.ds(sid * ROWS_PER_TILE, ROWS_PER_TILE)])


def _dinv_from_parts(parts):
    deg = jnp.sum(parts, axis=0)[:, None] + 1.0
    rid = lax.broadcasted_iota(jnp.int32, (NP, 1), 0)
    return jnp.where(rid < N_NODES, lax.rsqrt(deg), 0.0)


def _tc_first(x_ref, w_ref, parts_ref, g_ref):
    dinv = _dinv_from_parts(parts_ref[...])
    h = jnp.dot(x_ref[...], w_ref[...], preferred_element_type=jnp.float32)
    g_ref[...] = h * dinv


def _tc_mid(acc_ref, parts_ref, g_ref, b_ref, w_ref, out_ref):
    dinv = _dinv_from_parts(parts_ref[...])
    a = acc_ref[...]
    s = a[0] + a[1] + g_ref[...]
    h = jnp.maximum(s * dinv + b_ref[...], 0.0)
    out_ref[...] = jnp.dot(h, w_ref[...],
                           preferred_element_type=jnp.float32) * dinv


def _tc_final(acc_ref, parts_ref, g_ref, b_ref, batch_ref, wlin_ref, blin_ref,
              out_ref):
    dinv = _dinv_from_parts(parts_ref[...])
    a = acc_ref[...]
    s = a[0] + a[1] + g_ref[...]
    h = jnp.maximum(s * dinv + b_ref[...], 0.0)
    gid = lax.broadcasted_iota(jnp.int32, (N_GRAPHS, NP), 0)
    onehot = (gid == batch_ref[...]).astype(jnp.float32)
    summed = jnp.dot(onehot, h, preferred_element_type=jnp.float32)
    counts = jnp.sum(onehot, axis=1)[:, None]
    pooled = summed / jnp.maximum(counts, 1.0)
    out_ref[...] = (jnp.dot(pooled, wlin_ref[...],
                            preferred_element_type=jnp.float32)
                    + blin_ref[...])


_first_call = pl.pallas_call(
    _tc_first,
    out_shape=jax.ShapeDtypeStruct((NP, D), jnp.float32),
)

_mid_call = pl.pallas_call(
    _tc_mid,
    out_shape=jax.ShapeDtypeStruct((NP, D), jnp.float32),
)

_final_call = pl.pallas_call(
    _tc_final,
    out_shape=jax.ShapeDtypeStruct((N_GRAPHS, 1), jnp.float32),
)


def kernel(x, edge_index, batch, W1, b1, W2, b2, W3, b3, W_lin, b_lin):
    src = edge_index[0].astype(jnp.int32)
    dst = edge_index[1].astype(jnp.int32)
    pad_e = jnp.full((EP - N_EDGES,), N_NODES, jnp.int32)
    src2 = jnp.concatenate([src, pad_e]).reshape(EP // CHUNK, CHUNK)
    dst2 = jnp.concatenate([dst, pad_e]).reshape(EP // CHUNK, CHUNK)
    x_pad = jnp.concatenate(
        [x, jnp.zeros((NP - N_NODES, D), jnp.float32)])
    batch_pad = jnp.concatenate(
        [batch.astype(jnp.int32),
         jnp.full((NP - N_NODES,), N_GRAPHS, jnp.int32)]).reshape(1, NP)

    parts = _sc_degree(dst2)

    g = _first_call(x_pad, W1, parts)
    for (b, w) in ((b1, W2), (b2, W3)):
        acc = _sc_edge_scatter(g, src2, dst2)
        g = _mid_call(acc, parts, g, b.reshape(1, D), w)
    acc = _sc_edge_scatter(g, src2, dst2)
    out = _final_call(acc, parts, g, b3.reshape(1, D), batch_pad,
                      W_lin, b_lin.reshape(1, 1))
    return out.reshape(-1)
